# baseline (speedup 1.0000x reference)
import functools

import jax
import jax.numpy as jnp
from jax import lax
from jax.experimental import pallas as pl
from jax.experimental.pallas import tpu as pltpu
from jax.experimental.pallas import tpu_sc as plsc

E = 8
K = 2
D = 1024
H = 4096
T = 2048
BLK = 256
NALLOC = T * K + E * BLK
NB = NALLOC // BLK


def _gate_route_body(x_ref, wg_ref, bg_ref,
                     pos0_ref, pos1_ref, w0_ref, w1_ref,
                     be_ref, na_ref, chg_ref, pff_ref, pfe_ref):
    xv = x_ref[...]
    lt = lax.dot_general(wg_ref[...], xv, (((0,), (1,)), ((), ())),
                         preferred_element_type=jnp.float32)
    lt = lt + bg_ref[...]

    ti = lax.broadcasted_iota(jnp.int32, (E, T), 0)
    v0 = jnp.max(lt, axis=0, keepdims=True)
    i0 = jnp.min(jnp.where(lt == v0, ti, E), axis=0, keepdims=True)
    m1 = jnp.where(ti == i0, -jnp.inf, lt)
    v1 = jnp.max(m1, axis=0, keepdims=True)
    i1 = jnp.min(jnp.where(m1 == v1, ti, E), axis=0, keepdims=True)
    w0 = 1.0 / (1.0 + jnp.exp(v1 - v0))
    w1 = 1.0 - w0

    maskf = ((ti == i0) | (ti == i1)).astype(jnp.float32)
    ra = lax.broadcasted_iota(jnp.int32, (T, T), 0)
    ca = lax.broadcasted_iota(jnp.int32, (T, T), 1)
    ut = (ra < ca).astype(jnp.float32)
    csum = jnp.dot(maskf, ut, preferred_element_type=jnp.float32)

    counts = jnp.sum(maskf, axis=1, keepdims=True)
    ci = counts.astype(jnp.int32)
    padded = (((ci + (BLK - 1)) // BLK) * BLK).astype(jnp.float32)
    r8 = lax.broadcasted_iota(jnp.int32, (E, E), 0)
    c8 = lax.broadcasted_iota(jnp.int32, (E, E), 1)
    sl8 = (c8 < r8).astype(jnp.float32)
    off = jnp.dot(sl8, padded, preferred_element_type=jnp.float32)

    posmat = csum + off
    pos0 = jnp.sum(jnp.where(ti == i0, posmat, 0.0), axis=0)
    pos1 = jnp.sum(jnp.where(ti == i1, posmat, 0.0), axis=0)
    pos0_ref[...] = pos0.astype(jnp.int32)
    pos1_ref[...] = pos1.astype(jnp.int32)
    w0_ref[...] = w0[0, :]
    w1_ref[...] = w1[0, :]

    sb = off.astype(jnp.int32) // BLK
    nbec = padded.astype(jnp.int32) // BLK
    endb = sb + nbec
    natot = jnp.sum(padded).astype(jnp.int32) // BLK
    ib = lax.broadcasted_iota(jnp.int32, (E, NB), 1)
    i8 = lax.broadcasted_iota(jnp.int32, (E, 1), 0)
    be = jnp.sum((ib >= endb).astype(jnp.int32), axis=0)
    emax = jnp.max(jnp.where(padded > 0, i8, 0))
    be = jnp.minimum(be, emax)
    oh = be[None, :] == i8
    chg = jnp.sum((oh & (ib == sb)).astype(jnp.int32), axis=0)
    pff = jnp.sum((oh & (ib == endb - 1) & (endb < natot)).astype(jnp.int32),
                  axis=0)
    j8 = lax.broadcasted_iota(jnp.int32, (E, E), 1)
    k8 = lax.broadcasted_iota(jnp.int32, (E, E), 0)
    actdiag = jnp.where(j8 == k8, jnp.broadcast_to(padded, (E, E)), 0.0)
    act_row = jnp.dot(jnp.ones((1, E), jnp.float32), actdiag) > 0
    nxte = jnp.min(jnp.where((j8 > k8) & act_row, j8, E), axis=1,
                   keepdims=True)
    pfe = jnp.sum(jnp.where(oh, jnp.broadcast_to(nxte, (E, NB)), 0),
                  axis=0)
    be_ref[...] = be
    na_ref[...] = jnp.broadcast_to(natot, (1,))
    chg_ref[...] = chg
    pff_ref[...] = pff
    pfe_ref[...] = pfe


def _gate_route(x, Wg, bg):
    return pl.pallas_call(
        _gate_route_body,
        out_shape=[
            jax.ShapeDtypeStruct((T,), jnp.int32),
            jax.ShapeDtypeStruct((T,), jnp.int32),
            jax.ShapeDtypeStruct((T,), jnp.float32),
            jax.ShapeDtypeStruct((T,), jnp.float32),
            jax.ShapeDtypeStruct((NB,), jnp.int32),
            jax.ShapeDtypeStruct((1,), jnp.int32),
            jax.ShapeDtypeStruct((NB,), jnp.int32),
            jax.ShapeDtypeStruct((NB,), jnp.int32),
            jax.ShapeDtypeStruct((NB,), jnp.int32),
        ],
    )(x, Wg, bg.reshape(E, 1))


def _sc_scatter(x, pos0, pos1):
    info = plsc.get_sparse_core_info()
    nw = info.num_cores * info.num_subcores
    tw = T // nw
    mesh = plsc.VectorSubcoreMesh(core_axis_name="c", subcore_axis_name="s")

    @functools.partial(
        pl.kernel,
        mesh=mesh,
        out_type=jax.ShapeDtypeStruct((NALLOC, D), jnp.float32),
        scratch_types=[
            pltpu.VMEM((tw,), jnp.int32),
            pltpu.VMEM((tw,), jnp.int32),
            pltpu.VMEM((tw, D), jnp.float32),
            pltpu.SemaphoreType.DMA,
            pltpu.SemaphoreType.DMA,
        ],
    )
    def k(x_hbm, p0_hbm, p1_hbm, xs_hbm, i0_v, i1_v, rows_v, sem0, sem1):
        wid = lax.axis_index("s") * info.num_cores + lax.axis_index("c")
        base = wid * tw
        pltpu.sync_copy(p0_hbm.at[pl.ds(base, tw)], i0_v)
        pltpu.sync_copy(p1_hbm.at[pl.ds(base, tw)], i1_v)
        pltpu.sync_copy(x_hbm.at[pl.ds(base, tw)], rows_v)
        c0 = pltpu.async_copy(rows_v, xs_hbm.at[i0_v], sem0)
        c1 = pltpu.async_copy(rows_v, xs_hbm.at[i1_v], sem1)
        c0.wait()
        c1.wait()

    return k(x, pos0, pos1)


def _gelu_exact(a):
    return 0.5 * a * (1.0 + lax.erf(a * (2.0 ** -0.5)))


_NCH = 8


def _w_dma(w_hbm, e, stage, sem, major):
    cs = major // _NCH
    return [pltpu.make_async_copy(
        w_hbm.at[pl.ds(e, 1), pl.ds(c * cs, cs), :],
        stage.at[:, pl.ds(c * cs, cs), :], sem) for c in range(_NCH)]


def _ffn1_body(be_s, na_s, chg_s, pff_s, pfe_s,
               xs_ref, w1_hbm, b1_ref, h_ref, stage, w1b, sem):
    i = pl.program_id(0)

    @pl.when(i == 0)
    def _():
        for dsc in _w_dma(w1_hbm, be_s[0], stage, sem, D):
            dsc.start()

    @pl.when(chg_s[i] == 1)
    def _():
        for dsc in _w_dma(w1_hbm, be_s[i], stage, sem, D):
            dsc.wait()
        w1b[...] = stage[0].astype(jnp.bfloat16)

    @pl.when(pff_s[i] == 1)
    def _():
        for dsc in _w_dma(w1_hbm, pfe_s[i], stage, sem, D):
            dsc.start()

    @pl.when(i < na_s[0])
    def _():
        xb = xs_ref[...].astype(jnp.bfloat16)
        a = jnp.dot(xb, w1b[...], preferred_element_type=jnp.float32)
        h_ref[...] = _gelu_exact(a + b1_ref[0]).astype(jnp.bfloat16)


def _ffn1(ctl, x_sorted, W1, b1):
    be, na, chg, pff, pfe = ctl
    grid_spec = pltpu.PrefetchScalarGridSpec(
        num_scalar_prefetch=5,
        grid=(NB,),
        in_specs=[
            pl.BlockSpec((BLK, D),
                         lambda i, be, na, chg, pff, pfe:
                         (jnp.minimum(i, na[0] - 1), 0)),
            pl.BlockSpec(memory_space=pltpu.MemorySpace.HBM),
            pl.BlockSpec((1, 1, H),
                         lambda i, be, na, chg, pff, pfe: (be[i], 0, 0)),
        ],
        out_specs=pl.BlockSpec((BLK, H),
                               lambda i, be, na, chg, pff, pfe: (i, 0)),
        scratch_shapes=[
            pltpu.VMEM((1, D, H), jnp.float32),
            pltpu.VMEM((D, H), jnp.bfloat16),
            pltpu.SemaphoreType.DMA,
        ],
    )
    return pl.pallas_call(
        _ffn1_body,
        grid_spec=grid_spec,
        out_shape=jax.ShapeDtypeStruct((NALLOC, H), jnp.bfloat16),
    )(be, na, chg, pff, pfe, x_sorted, W1, b1.reshape(E, 1, H))


def _ffn2_body(be_s, na_s, chg_s, pff_s, pfe_s,
               h_ref, w2_hbm, b2_ref, y_ref, stage, w2b, sem):
    i = pl.program_id(0)

    @pl.when(i == 0)
    def _():
        for dsc in _w_dma(w2_hbm, be_s[0], stage, sem, H):
            dsc.start()

    @pl.when(chg_s[i] == 1)
    def _():
        for dsc in _w_dma(w2_hbm, be_s[i], stage, sem, H):
            dsc.wait()
        w2b[...] = stage[0].astype(jnp.bfloat16)

    @pl.when(pff_s[i] == 1)
    def _():
        for dsc in _w_dma(w2_hbm, pfe_s[i], stage, sem, H):
            dsc.start()

    @pl.when(i < na_s[0])
    def _():
        a = jnp.dot(h_ref[...], w2b[...], preferred_element_type=jnp.float32)
        y_ref[...] = a + b2_ref[0]


def _ffn2(ctl, h, W2, b2):
    be, na, chg, pff, pfe = ctl
    grid_spec = pltpu.PrefetchScalarGridSpec(
        num_scalar_prefetch=5,
        grid=(NB,),
        in_specs=[
            pl.BlockSpec((BLK, H),
                         lambda i, be, na, chg, pff, pfe:
                         (jnp.minimum(i, na[0] - 1), 0)),
            pl.BlockSpec(memory_space=pltpu.MemorySpace.HBM),
            pl.BlockSpec((1, 1, D),
                         lambda i, be, na, chg, pff, pfe: (be[i], 0, 0)),
        ],
        out_specs=pl.BlockSpec((BLK, D),
                               lambda i, be, na, chg, pff, pfe: (i, 0)),
        scratch_shapes=[
            pltpu.VMEM((1, H, D), jnp.float32),
            pltpu.VMEM((H, D), jnp.bfloat16),
            pltpu.SemaphoreType.DMA,
        ],
    )
    return pl.pallas_call(
        _ffn2_body,
        grid_spec=grid_spec,
        out_shape=jax.ShapeDtypeStruct((NALLOC, D), jnp.float32),
    )(be, na, chg, pff, pfe, h, W2, b2.reshape(E, 1, D))


def _sc_gather(y_sorted, pos0, pos1):
    info = plsc.get_sparse_core_info()
    nw = info.num_cores * info.num_subcores
    tw = T // nw
    mesh = plsc.VectorSubcoreMesh(core_axis_name="c", subcore_axis_name="s")

    @functools.partial(
        pl.kernel,
        mesh=mesh,
        out_type=[
            jax.ShapeDtypeStruct((T, D), jnp.float32),
            jax.ShapeDtypeStruct((T, D), jnp.float32),
        ],
        scratch_types=[
            pltpu.VMEM((tw,), jnp.int32),
            pltpu.VMEM((tw, D), jnp.float32),
            pltpu.SemaphoreType.DMA,
        ],
    )
    def k(y_hbm, p0_hbm, p1_hbm, y0_hbm, y1_hbm, idx_v, rows_v, sem):
        wid = lax.axis_index("s") * info.num_cores + lax.axis_index("c")
        base = wid * tw
        pltpu.sync_copy(p0_hbm.at[pl.ds(base, tw)], idx_v)
        pltpu.async_copy(y_hbm.at[idx_v], rows_v, sem).wait()
        pltpu.sync_copy(rows_v, y0_hbm.at[pl.ds(base, tw)])
        pltpu.sync_copy(p1_hbm.at[pl.ds(base, tw)], idx_v)
        pltpu.async_copy(y_hbm.at[idx_v], rows_v, sem).wait()
        pltpu.sync_copy(rows_v, y1_hbm.at[pl.ds(base, tw)])

    return k(y_sorted, pos0, pos1)


_CBLK = 256


def _combine_body(y0_ref, y1_ref, w0_ref, w1_ref, out_ref):
    ii = lax.broadcasted_iota(jnp.int32, (_CBLK, _CBLK), 0)
    jj = lax.broadcasted_iota(jnp.int32, (_CBLK, _CBLK), 1)
    d0 = jnp.where(ii == jj, jnp.broadcast_to(w0_ref[...], (_CBLK, _CBLK)), 0.0)
    d1 = jnp.where(ii == jj, jnp.broadcast_to(w1_ref[...], (_CBLK, _CBLK)), 0.0)
    out_ref[...] = (
        jnp.dot(d0, y0_ref[...], preferred_element_type=jnp.float32)
        + jnp.dot(d1, y1_ref[...], preferred_element_type=jnp.float32))


def _combine(y0, y1, w0, w1):
    return pl.pallas_call(
        _combine_body,
        grid=(T // _CBLK,),
        in_specs=[
            pl.BlockSpec((_CBLK, D), lambda i: (i, 0)),
            pl.BlockSpec((_CBLK, D), lambda i: (i, 0)),
            pl.BlockSpec((_CBLK,), lambda i: (i,)),
            pl.BlockSpec((_CBLK,), lambda i: (i,)),
        ],
        out_specs=pl.BlockSpec((_CBLK, D), lambda i: (i, 0)),
        out_shape=jax.ShapeDtypeStruct((T, D), jnp.float32),
    )(y0, y1, w0, w1)


def kernel(x, Wg, bg, W1, b1, W2, b2):
    xf = x.reshape(T, D)
    pos0, pos1, w0, w1, be, na, chg, pff, pfe = _gate_route(xf, Wg, bg)
    ctl = (be, na, chg, pff, pfe)
    x_sorted = _sc_scatter(xf, pos0, pos1)
    h = _ffn1(ctl, x_sorted, W1, b1)
    y_sorted = _ffn2(ctl, h, W2, b2)
    y0, y1 = _sc_gather(y_sorted, pos0, pos1)
    out = _combine(y0, y1, w0, w1)
    return out.reshape(x.shape)


# speedup vs baseline: 1.0655x; 1.0655x over previous
import functools

import jax
import jax.numpy as jnp
from jax import lax
from jax.experimental import pallas as pl
from jax.experimental.pallas import tpu as pltpu
from jax.experimental.pallas import tpu_sc as plsc

E = 8
K = 2
D = 1024
H = 4096
T = 2048
BLK = 512
NALLOC = T * K + E * BLK
NB = NALLOC // BLK


def _gate_route_body(x_ref, wg_ref, bg_ref,
                     pos0_ref, pos1_ref, w0_ref, w1_ref,
                     be_ref, na_ref, chg_ref, pff_ref, pfe_ref):
    xv = x_ref[...]
    lt = lax.dot_general(wg_ref[...], xv, (((0,), (1,)), ((), ())),
                         preferred_element_type=jnp.float32)
    lt = lt + bg_ref[...]

    ti = lax.broadcasted_iota(jnp.int32, (E, T), 0)
    v0 = jnp.max(lt, axis=0, keepdims=True)
    i0 = jnp.min(jnp.where(lt == v0, ti, E), axis=0, keepdims=True)
    m1 = jnp.where(ti == i0, -jnp.inf, lt)
    v1 = jnp.max(m1, axis=0, keepdims=True)
    i1 = jnp.min(jnp.where(m1 == v1, ti, E), axis=0, keepdims=True)
    w0 = 1.0 / (1.0 + jnp.exp(v1 - v0))
    w1 = 1.0 - w0

    maskf = ((ti == i0) | (ti == i1)).astype(jnp.float32)
    ra = lax.broadcasted_iota(jnp.int32, (T, T), 0)
    ca = lax.broadcasted_iota(jnp.int32, (T, T), 1)
    ut = (ra < ca).astype(jnp.float32)
    csum = jnp.dot(maskf, ut, preferred_element_type=jnp.float32)

    counts = jnp.sum(maskf, axis=1, keepdims=True)
    ci = counts.astype(jnp.int32)
    padded = (((ci + (BLK - 1)) // BLK) * BLK).astype(jnp.float32)
    r8 = lax.broadcasted_iota(jnp.int32, (E, E), 0)
    c8 = lax.broadcasted_iota(jnp.int32, (E, E), 1)
    sl8 = (c8 < r8).astype(jnp.float32)
    off = jnp.dot(sl8, padded, preferred_element_type=jnp.float32)

    posmat = csum + off
    pos0 = jnp.sum(jnp.where(ti == i0, posmat, 0.0), axis=0)
    pos1 = jnp.sum(jnp.where(ti == i1, posmat, 0.0), axis=0)
    pos0_ref[...] = pos0.astype(jnp.int32)
    pos1_ref[...] = pos1.astype(jnp.int32)
    w0_ref[...] = w0[0, :]
    w1_ref[...] = w1[0, :]

    sb = off.astype(jnp.int32) // BLK
    nbec = padded.astype(jnp.int32) // BLK
    endb = sb + nbec
    natot = jnp.sum(padded).astype(jnp.int32) // BLK
    ib = lax.broadcasted_iota(jnp.int32, (E, NB), 1)
    i8 = lax.broadcasted_iota(jnp.int32, (E, 1), 0)
    be = jnp.sum((ib >= endb).astype(jnp.int32), axis=0)
    emax = jnp.max(jnp.where(padded > 0, i8, 0))
    be = jnp.minimum(be, emax)
    oh = be[None, :] == i8
    chg = jnp.sum((oh & (ib == sb)).astype(jnp.int32), axis=0)
    pff = jnp.sum((oh & (ib == endb - 1) & (endb < natot)).astype(jnp.int32),
                  axis=0)
    j8 = lax.broadcasted_iota(jnp.int32, (E, E), 1)
    k8 = lax.broadcasted_iota(jnp.int32, (E, E), 0)
    actdiag = jnp.where(j8 == k8, jnp.broadcast_to(padded, (E, E)), 0.0)
    act_row = jnp.dot(jnp.ones((1, E), jnp.float32), actdiag) > 0
    nxte = jnp.min(jnp.where((j8 > k8) & act_row, j8, E), axis=1,
                   keepdims=True)
    pfe = jnp.sum(jnp.where(oh, jnp.broadcast_to(nxte, (E, NB)), 0),
                  axis=0)
    be_ref[...] = be
    na_ref[...] = jnp.broadcast_to(natot, (1,))
    chg_ref[...] = chg
    pff_ref[...] = pff
    pfe_ref[...] = pfe


def _gate_route(x, Wg, bg):
    return pl.pallas_call(
        _gate_route_body,
        out_shape=[
            jax.ShapeDtypeStruct((T,), jnp.int32),
            jax.ShapeDtypeStruct((T,), jnp.int32),
            jax.ShapeDtypeStruct((T,), jnp.float32),
            jax.ShapeDtypeStruct((T,), jnp.float32),
            jax.ShapeDtypeStruct((NB,), jnp.int32),
            jax.ShapeDtypeStruct((1,), jnp.int32),
            jax.ShapeDtypeStruct((NB,), jnp.int32),
            jax.ShapeDtypeStruct((NB,), jnp.int32),
            jax.ShapeDtypeStruct((NB,), jnp.int32),
        ],
    )(x, Wg, bg.reshape(E, 1))


def _sc_scatter(x, pos0, pos1):
    info = plsc.get_sparse_core_info()
    nw = info.num_cores * info.num_subcores
    tw = T // nw
    mesh = plsc.VectorSubcoreMesh(core_axis_name="c", subcore_axis_name="s")

    @functools.partial(
        pl.kernel,
        mesh=mesh,
        out_type=jax.ShapeDtypeStruct((NALLOC, D), jnp.float32),
        scratch_types=[
            pltpu.VMEM((tw,), jnp.int32),
            pltpu.VMEM((tw,), jnp.int32),
            pltpu.VMEM((tw, D), jnp.float32),
            pltpu.SemaphoreType.DMA,
            pltpu.SemaphoreType.DMA,
        ],
    )
    def k(x_hbm, p0_hbm, p1_hbm, xs_hbm, i0_v, i1_v, rows_v, sem0, sem1):
        wid = lax.axis_index("s") * info.num_cores + lax.axis_index("c")
        base = wid * tw
        pltpu.sync_copy(p0_hbm.at[pl.ds(base, tw)], i0_v)
        pltpu.sync_copy(p1_hbm.at[pl.ds(base, tw)], i1_v)
        pltpu.sync_copy(x_hbm.at[pl.ds(base, tw)], rows_v)
        c0 = pltpu.async_copy(rows_v, xs_hbm.at[i0_v], sem0)
        c1 = pltpu.async_copy(rows_v, xs_hbm.at[i1_v], sem1)
        c0.wait()
        c1.wait()

    return k(x, pos0, pos1)


def _gelu_exact(a):
    return 0.5 * a * (1.0 + lax.erf(a * (2.0 ** -0.5)))


_NCH = 8


def _w_dma(w_hbm, e, stage, sem, major):
    cs = major // _NCH
    return [pltpu.make_async_copy(
        w_hbm.at[pl.ds(e, 1), pl.ds(c * cs, cs), :],
        stage.at[:, pl.ds(c * cs, cs), :], sem) for c in range(_NCH)]


def _ffn1_body(be_s, na_s, chg_s, pff_s, pfe_s,
               xs_ref, w1_hbm, b1_ref, h_ref, stage, w1b, sem):
    i = pl.program_id(0)

    @pl.when(i == 0)
    def _():
        for dsc in _w_dma(w1_hbm, be_s[0], stage, sem, D):
            dsc.start()

    @pl.when(chg_s[i] == 1)
    def _():
        for dsc in _w_dma(w1_hbm, be_s[i], stage, sem, D):
            dsc.wait()
        w1b[...] = stage[0].astype(jnp.bfloat16)

    @pl.when(pff_s[i] == 1)
    def _():
        for dsc in _w_dma(w1_hbm, pfe_s[i], stage, sem, D):
            dsc.start()

    @pl.when(i < na_s[0])
    def _():
        xb = xs_ref[...].astype(jnp.bfloat16)
        a = jnp.dot(xb, w1b[...], preferred_element_type=jnp.float32)
        h_ref[...] = _gelu_exact(a + b1_ref[0]).astype(jnp.bfloat16)


def _ffn1(ctl, x_sorted, W1, b1):
    be, na, chg, pff, pfe = ctl
    grid_spec = pltpu.PrefetchScalarGridSpec(
        num_scalar_prefetch=5,
        grid=(NB,),
        in_specs=[
            pl.BlockSpec((BLK, D),
                         lambda i, be, na, chg, pff, pfe:
                         (jnp.minimum(i, na[0] - 1), 0)),
            pl.BlockSpec(memory_space=pltpu.MemorySpace.HBM),
            pl.BlockSpec((1, 1, H),
                         lambda i, be, na, chg, pff, pfe: (be[i], 0, 0)),
        ],
        out_specs=pl.BlockSpec((BLK, H),
                               lambda i, be, na, chg, pff, pfe: (i, 0)),
        scratch_shapes=[
            pltpu.VMEM((1, D, H), jnp.float32),
            pltpu.VMEM((D, H), jnp.bfloat16),
            pltpu.SemaphoreType.DMA,
        ],
    )
    return pl.pallas_call(
        _ffn1_body,
        grid_spec=grid_spec,
        out_shape=jax.ShapeDtypeStruct((NALLOC, H), jnp.bfloat16),
    )(be, na, chg, pff, pfe, x_sorted, W1, b1.reshape(E, 1, H))


def _ffn2_body(be_s, na_s, chg_s, pff_s, pfe_s,
               h_ref, w2_hbm, b2_ref, y_ref, stage, w2b, sem):
    i = pl.program_id(0)

    @pl.when(i == 0)
    def _():
        for dsc in _w_dma(w2_hbm, be_s[0], stage, sem, H):
            dsc.start()

    @pl.when(chg_s[i] == 1)
    def _():
        for dsc in _w_dma(w2_hbm, be_s[i], stage, sem, H):
            dsc.wait()
        w2b[...] = stage[0].astype(jnp.bfloat16)

    @pl.when(pff_s[i] == 1)
    def _():
        for dsc in _w_dma(w2_hbm, pfe_s[i], stage, sem, H):
            dsc.start()

    @pl.when(i < na_s[0])
    def _():
        a = jnp.dot(h_ref[...], w2b[...], preferred_element_type=jnp.float32)
        y_ref[...] = a + b2_ref[0]


def _ffn2(ctl, h, W2, b2):
    be, na, chg, pff, pfe = ctl
    grid_spec = pltpu.PrefetchScalarGridSpec(
        num_scalar_prefetch=5,
        grid=(NB,),
        in_specs=[
            pl.BlockSpec((BLK, H),
                         lambda i, be, na, chg, pff, pfe:
                         (jnp.minimum(i, na[0] - 1), 0)),
            pl.BlockSpec(memory_space=pltpu.MemorySpace.HBM),
            pl.BlockSpec((1, 1, D),
                         lambda i, be, na, chg, pff, pfe: (be[i], 0, 0)),
        ],
        out_specs=pl.BlockSpec((BLK, D),
                               lambda i, be, na, chg, pff, pfe: (i, 0)),
        scratch_shapes=[
            pltpu.VMEM((1, H, D), jnp.float32),
            pltpu.VMEM((H, D), jnp.bfloat16),
            pltpu.SemaphoreType.DMA,
        ],
    )
    return pl.pallas_call(
        _ffn2_body,
        grid_spec=grid_spec,
        out_shape=jax.ShapeDtypeStruct((NALLOC, D), jnp.float32),
    )(be, na, chg, pff, pfe, h, W2, b2.reshape(E, 1, D))


def _sc_gather(y_sorted, pos0, pos1):
    info = plsc.get_sparse_core_info()
    nw = info.num_cores * info.num_subcores
    tw = T // nw
    mesh = plsc.VectorSubcoreMesh(core_axis_name="c", subcore_axis_name="s")

    @functools.partial(
        pl.kernel,
        mesh=mesh,
        out_type=[
            jax.ShapeDtypeStruct((T, D), jnp.float32),
            jax.ShapeDtypeStruct((T, D), jnp.float32),
        ],
        scratch_types=[
            pltpu.VMEM((tw,), jnp.int32),
            pltpu.VMEM((tw, D), jnp.float32),
            pltpu.SemaphoreType.DMA,
        ],
    )
    def k(y_hbm, p0_hbm, p1_hbm, y0_hbm, y1_hbm, idx_v, rows_v, sem):
        wid = lax.axis_index("s") * info.num_cores + lax.axis_index("c")
        base = wid * tw
        pltpu.sync_copy(p0_hbm.at[pl.ds(base, tw)], idx_v)
        pltpu.async_copy(y_hbm.at[idx_v], rows_v, sem).wait()
        pltpu.sync_copy(rows_v, y0_hbm.at[pl.ds(base, tw)])
        pltpu.sync_copy(p1_hbm.at[pl.ds(base, tw)], idx_v)
        pltpu.async_copy(y_hbm.at[idx_v], rows_v, sem).wait()
        pltpu.sync_copy(rows_v, y1_hbm.at[pl.ds(base, tw)])

    return k(y_sorted, pos0, pos1)


_CBLK = 256


def _combine_body(y0_ref, y1_ref, w0_ref, w1_ref, out_ref):
    ii = lax.broadcasted_iota(jnp.int32, (_CBLK, _CBLK), 0)
    jj = lax.broadcasted_iota(jnp.int32, (_CBLK, _CBLK), 1)
    d0 = jnp.where(ii == jj, jnp.broadcast_to(w0_ref[...], (_CBLK, _CBLK)), 0.0)
    d1 = jnp.where(ii == jj, jnp.broadcast_to(w1_ref[...], (_CBLK, _CBLK)), 0.0)
    out_ref[...] = (
        jnp.dot(d0, y0_ref[...], preferred_element_type=jnp.float32)
        + jnp.dot(d1, y1_ref[...], preferred_element_type=jnp.float32))


def _combine(y0, y1, w0, w1):
    return pl.pallas_call(
        _combine_body,
        grid=(T // _CBLK,),
        in_specs=[
            pl.BlockSpec((_CBLK, D), lambda i: (i, 0)),
            pl.BlockSpec((_CBLK, D), lambda i: (i, 0)),
            pl.BlockSpec((_CBLK,), lambda i: (i,)),
            pl.BlockSpec((_CBLK,), lambda i: (i,)),
        ],
        out_specs=pl.BlockSpec((_CBLK, D), lambda i: (i, 0)),
        out_shape=jax.ShapeDtypeStruct((T, D), jnp.float32),
    )(y0, y1, w0, w1)


def kernel(x, Wg, bg, W1, b1, W2, b2):
    xf = x.reshape(T, D)
    pos0, pos1, w0, w1, be, na, chg, pff, pfe = _gate_route(xf, Wg, bg)
    ctl = (be, na, chg, pff, pfe)
    x_sorted = _sc_scatter(xf, pos0, pos1)
    h = _ffn1(ctl, x_sorted, W1, b1)
    y_sorted = _ffn2(ctl, h, W2, b2)
    y0, y1 = _sc_gather(y_sorted, pos0, pos1)
    out = _combine(y0, y1, w0, w1)
    return out.reshape(x.shape)


# speedup vs baseline: 1.1175x; 1.0487x over previous
import functools

import jax
import jax.numpy as jnp
from jax import lax
from jax.experimental import pallas as pl
from jax.experimental.pallas import tpu as pltpu
from jax.experimental.pallas import tpu_sc as plsc

E = 8
K = 2
D = 1024
H = 4096
T = 2048
BLK = 512
NALLOC = T * K + E * BLK
NB = NALLOC // BLK


def _gate_route_body(x_ref, wg_ref, bg_ref,
                     pos0_ref, pos1_ref, w0_ref, w1_ref,
                     be_ref, na_ref, chg_ref, pff_ref, pfe_ref):
    xv = x_ref[...]
    lt = lax.dot_general(wg_ref[...], xv, (((0,), (1,)), ((), ())),
                         preferred_element_type=jnp.float32)
    lt = lt + bg_ref[...]

    ti = lax.broadcasted_iota(jnp.int32, (E, T), 0)
    v0 = jnp.max(lt, axis=0, keepdims=True)
    i0 = jnp.min(jnp.where(lt == v0, ti, E), axis=0, keepdims=True)
    m1 = jnp.where(ti == i0, -jnp.inf, lt)
    v1 = jnp.max(m1, axis=0, keepdims=True)
    i1 = jnp.min(jnp.where(m1 == v1, ti, E), axis=0, keepdims=True)
    w0 = 1.0 / (1.0 + jnp.exp(v1 - v0))
    w1 = 1.0 - w0

    maskf = ((ti == i0) | (ti == i1)).astype(jnp.float32)
    ra = lax.broadcasted_iota(jnp.int32, (T, T), 0)
    ca = lax.broadcasted_iota(jnp.int32, (T, T), 1)
    ut = (ra < ca).astype(jnp.float32)
    csum = jnp.dot(maskf, ut, preferred_element_type=jnp.float32)

    counts = jnp.sum(maskf, axis=1, keepdims=True)
    ci = counts.astype(jnp.int32)
    padded = (((ci + (BLK - 1)) // BLK) * BLK).astype(jnp.float32)
    r8 = lax.broadcasted_iota(jnp.int32, (E, E), 0)
    c8 = lax.broadcasted_iota(jnp.int32, (E, E), 1)
    sl8 = (c8 < r8).astype(jnp.float32)
    off = jnp.dot(sl8, padded, preferred_element_type=jnp.float32)

    posmat = csum + off
    pos0 = jnp.sum(jnp.where(ti == i0, posmat, 0.0), axis=0)
    pos1 = jnp.sum(jnp.where(ti == i1, posmat, 0.0), axis=0)
    pos0_ref[...] = pos0.astype(jnp.int32)
    pos1_ref[...] = pos1.astype(jnp.int32)
    w0_ref[...] = w0[0, :]
    w1_ref[...] = w1[0, :]

    sb = off.astype(jnp.int32) // BLK
    nbec = padded.astype(jnp.int32) // BLK
    endb = sb + nbec
    natot = jnp.sum(padded).astype(jnp.int32) // BLK
    ib = lax.broadcasted_iota(jnp.int32, (E, NB), 1)
    i8 = lax.broadcasted_iota(jnp.int32, (E, 1), 0)
    be = jnp.sum((ib >= endb).astype(jnp.int32), axis=0)
    emax = jnp.max(jnp.where(padded > 0, i8, 0))
    be = jnp.minimum(be, emax)
    oh = be[None, :] == i8
    chg = jnp.sum((oh & (ib == sb)).astype(jnp.int32), axis=0)
    pff = jnp.sum((oh & (ib == endb - 1) & (endb < natot)).astype(jnp.int32),
                  axis=0)
    j8 = lax.broadcasted_iota(jnp.int32, (E, E), 1)
    k8 = lax.broadcasted_iota(jnp.int32, (E, E), 0)
    actdiag = jnp.where(j8 == k8, jnp.broadcast_to(padded, (E, E)), 0.0)
    act_row = jnp.dot(jnp.ones((1, E), jnp.float32), actdiag) > 0
    nxte = jnp.min(jnp.where((j8 > k8) & act_row, j8, E), axis=1,
                   keepdims=True)
    pfe = jnp.sum(jnp.where(oh, jnp.broadcast_to(nxte, (E, NB)), 0),
                  axis=0)
    be_ref[...] = be
    na_ref[...] = jnp.broadcast_to(natot, (1,))
    chg_ref[...] = chg
    pff_ref[...] = pff
    pfe_ref[...] = pfe


def _gate_route(x, Wg, bg):
    return pl.pallas_call(
        _gate_route_body,
        out_shape=[
            jax.ShapeDtypeStruct((T,), jnp.int32),
            jax.ShapeDtypeStruct((T,), jnp.int32),
            jax.ShapeDtypeStruct((T,), jnp.float32),
            jax.ShapeDtypeStruct((T,), jnp.float32),
            jax.ShapeDtypeStruct((NB,), jnp.int32),
            jax.ShapeDtypeStruct((1,), jnp.int32),
            jax.ShapeDtypeStruct((NB,), jnp.int32),
            jax.ShapeDtypeStruct((NB,), jnp.int32),
            jax.ShapeDtypeStruct((NB,), jnp.int32),
        ],
    )(x, Wg, bg.reshape(E, 1))


def _sc_scatter(x, pos0, pos1):
    info = plsc.get_sparse_core_info()
    nw = info.num_cores * info.num_subcores
    tw = T // nw
    mesh = plsc.VectorSubcoreMesh(core_axis_name="c", subcore_axis_name="s")

    @functools.partial(
        pl.kernel,
        mesh=mesh,
        out_type=jax.ShapeDtypeStruct((NALLOC, D), jnp.float32),
        scratch_types=[
            pltpu.VMEM((tw,), jnp.int32),
            pltpu.VMEM((tw,), jnp.int32),
            pltpu.VMEM((tw, D), jnp.float32),
            pltpu.SemaphoreType.DMA,
            pltpu.SemaphoreType.DMA,
        ],
    )
    def k(x_hbm, p0_hbm, p1_hbm, xs_hbm, i0_v, i1_v, rows_v, sem0, sem1):
        wid = lax.axis_index("s") * info.num_cores + lax.axis_index("c")
        base = wid * tw
        pltpu.sync_copy(p0_hbm.at[pl.ds(base, tw)], i0_v)
        pltpu.sync_copy(p1_hbm.at[pl.ds(base, tw)], i1_v)
        pltpu.sync_copy(x_hbm.at[pl.ds(base, tw)], rows_v)
        c0 = pltpu.async_copy(rows_v, xs_hbm.at[i0_v], sem0)
        c1 = pltpu.async_copy(rows_v, xs_hbm.at[i1_v], sem1)
        c0.wait()
        c1.wait()

    return k(x, pos0, pos1)


def _gelu_exact(a):
    return 0.5 * a * (1.0 + lax.erf(a * (2.0 ** -0.5)))


_NCH = 8


def _w_dma(w_hbm, e, stage, sem, major):
    cs = major // _NCH
    return [pltpu.make_async_copy(
        w_hbm.at[pl.ds(e, 1), pl.ds(c * cs, cs), :],
        stage.at[:, pl.ds(c * cs, cs), :], sem) for c in range(_NCH)]


def _ffn1_body(be_s, na_s, chg_s, pff_s, pfe_s,
               xs_ref, w1_ref, b1_ref, h_ref):
    i = pl.program_id(0)

    @pl.when(i < na_s[0])
    def _():
        xb = xs_ref[...].astype(jnp.bfloat16)
        w1b = w1_ref[0].astype(jnp.bfloat16)
        a = jnp.dot(xb, w1b, preferred_element_type=jnp.float32)
        h_ref[...] = _gelu_exact(a + b1_ref[0]).astype(jnp.bfloat16)


def _ffn1(ctl, x_sorted, W1, b1):
    be, na, chg, pff, pfe = ctl
    grid_spec = pltpu.PrefetchScalarGridSpec(
        num_scalar_prefetch=5,
        grid=(NB,),
        in_specs=[
            pl.BlockSpec((BLK, D),
                         lambda i, be, na, chg, pff, pfe:
                         (jnp.minimum(i, na[0] - 1), 0)),
            pl.BlockSpec((1, D, H),
                         lambda i, be, na, chg, pff, pfe: (be[i], 0, 0)),
            pl.BlockSpec((1, 1, H),
                         lambda i, be, na, chg, pff, pfe: (be[i], 0, 0)),
        ],
        out_specs=pl.BlockSpec((BLK, H),
                               lambda i, be, na, chg, pff, pfe: (i, 0)),
    )
    return pl.pallas_call(
        _ffn1_body,
        grid_spec=grid_spec,
        out_shape=jax.ShapeDtypeStruct((NALLOC, H), jnp.bfloat16),
    )(be, na, chg, pff, pfe, x_sorted, W1, b1.reshape(E, 1, H))


def _ffn2_body(be_s, na_s, chg_s, pff_s, pfe_s,
               h_ref, w2_ref, b2_ref, y_ref):
    i = pl.program_id(0)

    @pl.when(i < na_s[0])
    def _():
        w2b = w2_ref[0].astype(jnp.bfloat16)
        a = jnp.dot(h_ref[...], w2b, preferred_element_type=jnp.float32)
        y_ref[...] = a + b2_ref[0]


def _ffn2(ctl, h, W2, b2):
    be, na, chg, pff, pfe = ctl
    grid_spec = pltpu.PrefetchScalarGridSpec(
        num_scalar_prefetch=5,
        grid=(NB,),
        in_specs=[
            pl.BlockSpec((BLK, H),
                         lambda i, be, na, chg, pff, pfe:
                         (jnp.minimum(i, na[0] - 1), 0)),
            pl.BlockSpec((1, H, D),
                         lambda i, be, na, chg, pff, pfe: (be[i], 0, 0)),
            pl.BlockSpec((1, 1, D),
                         lambda i, be, na, chg, pff, pfe: (be[i], 0, 0)),
        ],
        out_specs=pl.BlockSpec((BLK, D),
                               lambda i, be, na, chg, pff, pfe: (i, 0)),
    )
    return pl.pallas_call(
        _ffn2_body,
        grid_spec=grid_spec,
        out_shape=jax.ShapeDtypeStruct((NALLOC, D), jnp.float32),
    )(be, na, chg, pff, pfe, h, W2, b2.reshape(E, 1, D))


def _sc_gather(y_sorted, pos0, pos1):
    info = plsc.get_sparse_core_info()
    nw = info.num_cores * info.num_subcores
    tw = T // nw
    mesh = plsc.VectorSubcoreMesh(core_axis_name="c", subcore_axis_name="s")

    @functools.partial(
        pl.kernel,
        mesh=mesh,
        out_type=[
            jax.ShapeDtypeStruct((T, D), jnp.float32),
            jax.ShapeDtypeStruct((T, D), jnp.float32),
        ],
        scratch_types=[
            pltpu.VMEM((tw,), jnp.int32),
            pltpu.VMEM((tw, D), jnp.float32),
            pltpu.SemaphoreType.DMA,
        ],
    )
    def k(y_hbm, p0_hbm, p1_hbm, y0_hbm, y1_hbm, idx_v, rows_v, sem):
        wid = lax.axis_index("s") * info.num_cores + lax.axis_index("c")
        base = wid * tw
        pltpu.sync_copy(p0_hbm.at[pl.ds(base, tw)], idx_v)
        pltpu.async_copy(y_hbm.at[idx_v], rows_v, sem).wait()
        pltpu.sync_copy(rows_v, y0_hbm.at[pl.ds(base, tw)])
        pltpu.sync_copy(p1_hbm.at[pl.ds(base, tw)], idx_v)
        pltpu.async_copy(y_hbm.at[idx_v], rows_v, sem).wait()
        pltpu.sync_copy(rows_v, y1_hbm.at[pl.ds(base, tw)])

    return k(y_sorted, pos0, pos1)


_CBLK = 256


def _combine_body(y0_ref, y1_ref, w0_ref, w1_ref, out_ref):
    ii = lax.broadcasted_iota(jnp.int32, (_CBLK, _CBLK), 0)
    jj = lax.broadcasted_iota(jnp.int32, (_CBLK, _CBLK), 1)
    d0 = jnp.where(ii == jj, jnp.broadcast_to(w0_ref[...], (_CBLK, _CBLK)), 0.0)
    d1 = jnp.where(ii == jj, jnp.broadcast_to(w1_ref[...], (_CBLK, _CBLK)), 0.0)
    out_ref[...] = (
        jnp.dot(d0, y0_ref[...], preferred_element_type=jnp.float32)
        + jnp.dot(d1, y1_ref[...], preferred_element_type=jnp.float32))


def _combine(y0, y1, w0, w1):
    return pl.pallas_call(
        _combine_body,
        grid=(T // _CBLK,),
        in_specs=[
            pl.BlockSpec((_CBLK, D), lambda i: (i, 0)),
            pl.BlockSpec((_CBLK, D), lambda i: (i, 0)),
            pl.BlockSpec((_CBLK,), lambda i: (i,)),
            pl.BlockSpec((_CBLK,), lambda i: (i,)),
        ],
        out_specs=pl.BlockSpec((_CBLK, D), lambda i: (i, 0)),
        out_shape=jax.ShapeDtypeStruct((T, D), jnp.float32),
    )(y0, y1, w0, w1)


def kernel(x, Wg, bg, W1, b1, W2, b2):
    xf = x.reshape(T, D)
    pos0, pos1, w0, w1, be, na, chg, pff, pfe = _gate_route(xf, Wg, bg)
    ctl = (be, na, chg, pff, pfe)
    x_sorted = _sc_scatter(xf, pos0, pos1)
    h = _ffn1(ctl, x_sorted, W1, b1)
    y_sorted = _ffn2(ctl, h, W2, b2)
    y0, y1 = _sc_gather(y_sorted, pos0, pos1)
    out = _combine(y0, y1, w0, w1)
    return out.reshape(x.shape)
